# baseline (speedup 1.0000x reference)
import functools

import jax
import jax.numpy as jnp
from jax import lax
from jax.experimental import pallas as pl
from jax.experimental.pallas import tpu as pltpu
from jax.experimental.pallas import tpu_sc as plsc

NC = 2
NS = 16
NW = NC * NS
LANES = 16
CH = 16
NBUF = 3


@functools.lru_cache(maxsize=None)
def _build(nb, seq, d):
    pw = seq // NW
    n_h = pw // CH
    n_chunks = n_h * nb
    mesh = plsc.VectorSubcoreMesh(
        core_axis_name="c", subcore_axis_name="s",
        num_cores=NC, num_subcores=NS)

    @functools.partial(
        pl.kernel,
        out_type=jax.ShapeDtypeStruct((nb * seq, d), jnp.float32),
        mesh=mesh,
        scratch_types=(
            [pltpu.VMEM((pw,), jnp.int32) for _ in range(nb)]
            + [pltpu.VMEM((CH, d), jnp.float32) for _ in range(NBUF)]
            + [pltpu.VMEM((pw, d), jnp.float32)]
            + [pltpu.SemaphoreType.DMA for _ in range(2 * NBUF + 2)]
        ),
    )
    def emb(ids_hbm, wte_hbm, wpe_hbm, out_hbm, *refs):
        idx = refs[:nb]
        tok = refs[nb:nb + NBUF]
        pos_v = refs[nb + NBUF]
        sg = refs[nb + NBUF + 1:nb + NBUF + 1 + NBUF]
        ss = refs[nb + NBUF + 1 + NBUF:nb + NBUF + 1 + 2 * NBUF]
        si = refs[nb + NBUF + 1 + 2 * NBUF]
        sp = refs[nb + NBUF + 2 + 2 * NBUF]
        wid = lax.axis_index("s") * NC + lax.axis_index("c")
        pbase = wid * pw

        def parts(k):
            return k // nb, k % nb

        g = [None] * NBUF
        s = [None] * NBUF

        def start_gather(k):
            h, b = parts(k)
            p = k % NBUF
            g[p] = pltpu.async_copy(
                wte_hbm.at[idx[b].at[pl.ds(h * CH, CH)]], tok[p], sg[p])

        def process(k):
            h, b = parts(k)
            q = k % NBUF
            g[q].wait()

            def body(r, carry):
                for i in range(d // LANES):
                    sl = pl.ds(i * LANES, LANES)
                    tok[q][r, sl] = tok[q][r, sl] + pos_v[h * CH + r, sl]
                return carry
            lax.fori_loop(0, CH, body, 0)
            s[q] = pltpu.async_copy(
                tok[q], out_hbm.at[pl.ds(b * seq + pbase + h * CH, CH)],
                ss[q])

        iws = [pltpu.async_copy(ids_hbm.at[b, pl.ds(pbase, pw)], idx[b], si)
               for b in range(nb)]
        pos_w = pltpu.async_copy(wpe_hbm.at[pl.ds(pbase, pw)], pos_v, sp)
        for iw in iws:
            iw.wait()

        start_gather(0)
        start_gather(1)
        pos_w.wait()
        for k in range(2, n_chunks):
            p = k % NBUF
            if s[p] is not None:
                s[p].wait()
            start_gather(k)
            process(k - 2)
        process(n_chunks - 2)
        process(n_chunks - 1)
        for p in range(NBUF):
            if s[p] is not None:
                s[p].wait()

    return emb


def kernel(input_ids, wte, wpe):
    b, s = input_ids.shape
    d = wte.shape[1]
    emb = _build(b, s, d)
    out = emb(input_ids, wte, wpe)
    return out.reshape(b, s, d)
